# baseline (speedup 1.0000x reference)
import functools

import jax
import jax.numpy as jnp
from jax import lax
from jax.experimental import pallas as pl
from jax.experimental.pallas import tpu as pltpu
from jax.experimental.pallas import tpu_sc as plsc

_NC = 2
_NS = 16
_NW = _NC * _NS

_GATHER_CHUNK = 64


def _sc_gather_rows(table, idx):
    n = idx.shape[0]
    d = table.shape[1]
    rpw = n // _NW
    ch = min(_GATHER_CHUNK, rpw)
    nch = rpw // ch
    assert rpw % ch == 0

    mesh = plsc.VectorSubcoreMesh(
        core_axis_name="c", subcore_axis_name="s",
        num_cores=_NC, num_subcores=_NS,
    )

    @functools.partial(
        pl.kernel,
        mesh=mesh,
        out_type=jax.ShapeDtypeStruct((n, d), jnp.float32),
        scratch_types=[
            pltpu.VMEM((rpw,), jnp.int32),
            pltpu.VMEM((ch, d), jnp.float32),
            pltpu.VMEM((ch, d), jnp.float32),
            pltpu.SemaphoreType.DMA,
            pltpu.SemaphoreType.DMA,
            pltpu.SemaphoreType.DMA,
            pltpu.SemaphoreType.DMA,
        ],
    )
    def k(table_hbm, idx_hbm, out_hbm, idx_v, rows0, rows1, g0, g1, s0, s1):
        wid = lax.axis_index("s") * _NC + lax.axis_index("c")
        base = wid * rpw
        pltpu.sync_copy(idx_hbm.at[pl.ds(base, rpw)], idx_v)
        rows = (rows0, rows1)
        gsem = (g0, g1)
        ssem = (s0, s1)

        def gather(ci):
            b = ci % 2
            return pltpu.async_copy(
                table_hbm.at[idx_v.at[pl.ds(ci * ch, ch)]], rows[b], gsem[b])

        def store(ci):
            b = ci % 2
            return pltpu.async_copy(
                rows[b], out_hbm.at[pl.ds(base + ci * ch, ch)], ssem[b])

        gcp = [None, None]
        scp = [None, None]
        gcp[0] = gather(0)
        for ci in range(nch):
            b = ci % 2
            nb = (ci + 1) % 2
            if ci + 1 < nch:
                if scp[nb] is not None:
                    scp[nb].wait()
                gcp[nb] = gather(ci + 1)
            gcp[b].wait()
            scp[b] = store(ci)
        for cp in scp:
            if cp is not None:
                cp.wait()

    return k(table, idx)


_TC_BLOCK_ROWS = 256


def _tc_add_layernorm_part(tok, pos_table, seg_pad, seg_f, gamma2d, beta2d,
                           eps, prev, off_blocks, total_n):
    p, d = tok.shape
    s = pos_table.shape[0]
    br = _TC_BLOCK_ROWS
    assert p % br == 0 and s % br == 0
    nblk = p // br
    sblk = s // br

    def body(tok_ref, pos_ref, seg_ref, sid_ref, g_ref, b_ref, *rest):
        o_ref = rest[-1]
        s0 = seg_ref[0, :]
        sd = seg_ref[1, :] - s0
        x = tok_ref[...] + pos_ref[...] + s0[None, :] + sid_ref[...] * sd[None, :]
        mean = jnp.mean(x, axis=-1, keepdims=True)
        xc = x - mean
        var = jnp.mean(xc * xc, axis=-1, keepdims=True)
        inv = lax.rsqrt(var + eps)
        o_ref[...] = xc * inv * g_ref[...] + b_ref[...]

    in_specs = [
        pl.BlockSpec((br, d), lambda i: (i, 0)),
        pl.BlockSpec((br, d), lambda i: ((i + off_blocks) % sblk, 0)),
        pl.BlockSpec((8, d), lambda i: (0, 0)),
        pl.BlockSpec((br, 1), lambda i: (i, 0)),
        pl.BlockSpec((1, d), lambda i: (0, 0)),
        pl.BlockSpec((1, d), lambda i: (0, 0)),
    ]
    args = [tok, pos_table, seg_pad, seg_f, gamma2d, beta2d]
    aliases = {}
    if prev is not None:
        in_specs.append(pl.BlockSpec(memory_space=pl.ANY))
        args.append(prev)
        aliases = {6: 0}

    return pl.pallas_call(
        body,
        grid=(nblk,),
        in_specs=in_specs,
        out_specs=pl.BlockSpec((br, d), lambda i: (i + off_blocks, 0)),
        out_shape=jax.ShapeDtypeStruct((total_n, d), jnp.float32),
        input_output_aliases=aliases,
    )(*args)


_NPARTS = 4


def kernel(token_ids, segment_ids, input_ids, token_table, segment_table,
           position_table, ln_gamma, ln_beta):
    b, s = input_ids.shape
    d = token_table.shape[1]
    n = b * s
    br = _TC_BLOCK_ROWS

    idx = token_ids.reshape(n).astype(jnp.int32)
    seg_pad = jnp.pad(segment_table, ((0, 8 - segment_table.shape[0]), (0, 0)))
    seg_f = segment_ids.reshape(n, 1).astype(jnp.float32)
    gamma2d = ln_gamma.reshape(1, d)
    beta2d = ln_beta.reshape(1, d)

    p = n // _NPARTS
    assert p % br == 0
    toks = [
        _sc_gather_rows(token_table, lax.slice(idx, (h * p,), ((h + 1) * p,)))
        for h in range(_NPARTS)
    ]
    out = None
    for h in range(_NPARTS):
        out = _tc_add_layernorm_part(
            toks[h], position_table, seg_pad,
            lax.slice(seg_f, (h * p, 0), ((h + 1) * p, 1)),
            gamma2d, beta2d, 1e-5, out, h * (p // br), n,
        )
    return out.reshape(b, s, d)


# speedup vs baseline: 1.0074x; 1.0074x over previous
import functools

import jax
import jax.numpy as jnp
from jax import lax
from jax.experimental import pallas as pl
from jax.experimental.pallas import tpu as pltpu
from jax.experimental.pallas import tpu_sc as plsc

_NC = 2
_NS = 16
_NW = _NC * _NS

_NPARTS = 4
_TC_BLOCK_ROWS = 256
_GATHER_CHUNK = 32


def _sc_gather_part(table, token_ids, h, s_chunk):
    b_sz, s_len = token_ids.shape
    d = table.shape[1]
    n_p = b_sz * s_chunk
    rpw = n_p // _NW
    wpb = _NW // b_sz
    ch = min(_GATHER_CHUNK, rpw)
    nch = rpw // ch
    assert rpw * _NW == n_p and ch * nch == rpw

    mesh = plsc.VectorSubcoreMesh(
        core_axis_name="c", subcore_axis_name="s",
        num_cores=_NC, num_subcores=_NS,
    )

    @functools.partial(
        pl.kernel,
        mesh=mesh,
        out_type=jax.ShapeDtypeStruct((n_p, d), jnp.float32),
        scratch_types=[
            pltpu.VMEM((rpw,), jnp.int32),
            pltpu.VMEM((ch, d), jnp.float32),
            pltpu.VMEM((ch, d), jnp.float32),
            pltpu.SemaphoreType.DMA,
            pltpu.SemaphoreType.DMA,
            pltpu.SemaphoreType.DMA,
            pltpu.SemaphoreType.DMA,
        ],
    )
    def k(table_hbm, tids_hbm, out_hbm, idx_v, rows0, rows1, g0, g1, s0, s1):
        wid = lax.axis_index("s") * _NC + lax.axis_index("c")
        bi = wid // wpb
        si = h * s_chunk + (wid % wpb) * rpw
        base = wid * rpw
        pltpu.sync_copy(tids_hbm.at[bi, pl.ds(si, rpw)], idx_v)
        rows = (rows0, rows1)
        gsem = (g0, g1)
        ssem = (s0, s1)

        def gather(ci):
            b = ci % 2
            return pltpu.async_copy(
                table_hbm.at[idx_v.at[pl.ds(ci * ch, ch)]], rows[b], gsem[b])

        def store(ci):
            b = ci % 2
            return pltpu.async_copy(
                rows[b], out_hbm.at[pl.ds(base + ci * ch, ch)], ssem[b])

        gcp = [None, None]
        scp = [None, None]
        gcp[0] = gather(0)
        for ci in range(nch):
            b = ci % 2
            nb = (ci + 1) % 2
            if ci + 1 < nch:
                if scp[nb] is not None:
                    scp[nb].wait()
                gcp[nb] = gather(ci + 1)
            gcp[b].wait()
            scp[b] = store(ci)
        for cp in scp:
            if cp is not None:
                cp.wait()

    return k(table, token_ids)


def _tc_add_layernorm_part(tok, pos_table, seg_pad, seg_f, gamma2d, beta2d,
                           eps, prev, h, s_chunk, total_n):
    n_p, d = tok.shape
    s_len = pos_table.shape[0]
    br = _TC_BLOCK_ROWS
    assert n_p % br == 0 and s_chunk % br == 0
    bps = s_chunk // br
    spb = s_len // br
    nblk = n_p // br

    def out_map(i):
        return ((i // bps) * spb + h * bps + (i % bps), 0)

    def body(tok_ref, pos_ref, seg_ref, sid_ref, g_ref, b_ref, *rest):
        o_ref = rest[-1]
        s0 = seg_ref[0, :]
        sd = seg_ref[1, :] - s0
        x = tok_ref[...] + pos_ref[...] + s0[None, :] + sid_ref[...] * sd[None, :]
        mean = jnp.mean(x, axis=-1, keepdims=True)
        xc = x - mean
        var = jnp.mean(xc * xc, axis=-1, keepdims=True)
        inv = lax.rsqrt(var + eps)
        o_ref[...] = xc * inv * g_ref[...] + b_ref[...]

    in_specs = [
        pl.BlockSpec((br, d), lambda i: (i, 0)),
        pl.BlockSpec((br, d), lambda i: (h * bps + (i % bps), 0)),
        pl.BlockSpec((8, d), lambda i: (0, 0)),
        pl.BlockSpec((br, 1), out_map),
        pl.BlockSpec((1, d), lambda i: (0, 0)),
        pl.BlockSpec((1, d), lambda i: (0, 0)),
    ]
    args = [tok, pos_table, seg_pad, seg_f, gamma2d, beta2d]
    aliases = {}
    if prev is not None:
        in_specs.append(pl.BlockSpec(memory_space=pl.ANY))
        args.append(prev)
        aliases = {6: 0}

    return pl.pallas_call(
        body,
        grid=(nblk,),
        in_specs=in_specs,
        out_specs=pl.BlockSpec((br, d), out_map),
        out_shape=jax.ShapeDtypeStruct((total_n, d), jnp.float32),
        input_output_aliases=aliases,
    )(*args)


def kernel(token_ids, segment_ids, input_ids, token_table, segment_table,
           position_table, ln_gamma, ln_beta):
    b, s = input_ids.shape
    d = token_table.shape[1]
    n = b * s

    tids = token_ids.astype(jnp.int32)
    seg_pad = jnp.pad(segment_table, ((0, 8 - segment_table.shape[0]), (0, 0)))
    seg_f = segment_ids.reshape(n, 1).astype(jnp.float32)
    gamma2d = ln_gamma.reshape(1, d)
    beta2d = ln_beta.reshape(1, d)

    s_chunk = s // _NPARTS
    toks = [_sc_gather_part(token_table, tids, h, s_chunk)
            for h in range(_NPARTS)]
    out = None
    for h in range(_NPARTS):
        out = _tc_add_layernorm_part(
            toks[h], position_table, seg_pad, seg_f, gamma2d, beta2d,
            1e-5, out, h, s_chunk, n,
        )
    return out.reshape(b, s, d)


# speedup vs baseline: 1.0972x; 1.0892x over previous
import functools

import jax
import jax.numpy as jnp
from jax import lax
from jax.experimental import pallas as pl
from jax.experimental.pallas import tpu as pltpu
from jax.experimental.pallas import tpu_sc as plsc

_NC = 2
_NS = 16
_NW = _NC * _NS

_NPARTS = 4
_TC_BLOCK_ROWS = 256
_GATHER_CHUNK = 32


def _sc_gather_part(table, tids_part):
    b_sz, s_chunk = tids_part.shape
    d = table.shape[1]
    n_p = b_sz * s_chunk
    rpw = n_p // _NW
    wpb = _NW // b_sz
    ch = min(_GATHER_CHUNK, rpw)
    nch = rpw // ch
    assert rpw * _NW == n_p and ch * nch == rpw

    mesh = plsc.VectorSubcoreMesh(
        core_axis_name="c", subcore_axis_name="s",
        num_cores=_NC, num_subcores=_NS,
    )

    @functools.partial(
        pl.kernel,
        mesh=mesh,
        out_type=jax.ShapeDtypeStruct((n_p, d), jnp.float32),
        scratch_types=[
            pltpu.VMEM((rpw,), jnp.int32),
            pltpu.VMEM((ch, d), jnp.float32),
            pltpu.VMEM((ch, d), jnp.float32),
            pltpu.SemaphoreType.DMA,
            pltpu.SemaphoreType.DMA,
            pltpu.SemaphoreType.DMA,
            pltpu.SemaphoreType.DMA,
        ],
    )
    def k(table_hbm, tids_hbm, out_hbm, idx_v, rows0, rows1, g0, g1, s0, s1):
        wid = lax.axis_index("s") * _NC + lax.axis_index("c")
        bi = wid // wpb
        si = (wid % wpb) * rpw
        base = wid * rpw
        pltpu.sync_copy(tids_hbm.at[bi, pl.ds(si, rpw)], idx_v)
        rows = (rows0, rows1)
        gsem = (g0, g1)
        ssem = (s0, s1)

        def gather(ci):
            b = ci % 2
            return pltpu.async_copy(
                table_hbm.at[idx_v.at[pl.ds(ci * ch, ch)]], rows[b], gsem[b])

        def store(ci):
            b = ci % 2
            return pltpu.async_copy(
                rows[b], out_hbm.at[pl.ds(base + ci * ch, ch)], ssem[b])

        gcp = [None, None]
        scp = [None, None]
        gcp[0] = gather(0)
        for ci in range(nch):
            b = ci % 2
            nb = (ci + 1) % 2
            if ci + 1 < nch:
                if scp[nb] is not None:
                    scp[nb].wait()
                gcp[nb] = gather(ci + 1)
            gcp[b].wait()
            scp[b] = store(ci)
        for cp in scp:
            if cp is not None:
                cp.wait()

    return k(table, tids_part)


def _tc_add_layernorm_part(tok, pos_table, seg_table, sid3, gamma2d, beta2d,
                           eps, prev, h, s_chunk, total_n, b_sz):
    n_p, d = tok.shape
    s_len = pos_table.shape[0]
    br = _TC_BLOCK_ROWS
    assert n_p % br == 0 and s_chunk % br == 0
    bps = s_chunk // br
    spb = s_len // br
    nblk = n_p // br

    def out_map(i):
        return ((i % b_sz) * spb + h * bps + i // b_sz, 0)

    def body(tok_ref, pos_ref, seg_ref, sid_ref, g_ref, b_ref, *rest):
        o_ref = rest[-1]
        s0 = seg_ref[0, :]
        sd = seg_ref[1, :] - s0
        sid_col = sid_ref[0, 0, :].astype(jnp.float32).reshape(br, 1)
        x = tok_ref[...] + pos_ref[...] + s0[None, :] + sid_col * sd[None, :]
        mean = jnp.mean(x, axis=-1, keepdims=True)
        xc = x - mean
        var = jnp.mean(xc * xc, axis=-1, keepdims=True)
        inv = lax.rsqrt(var + eps)
        o_ref[...] = xc * inv * g_ref[...] + b_ref[...]

    in_specs = [
        pl.BlockSpec((br, d), lambda i: ((i % b_sz) * bps + i // b_sz, 0)),
        pl.BlockSpec((br, d), lambda i: (h * bps + i // b_sz, 0)),
        pl.BlockSpec((8, d), lambda i: (0, 0)),
        pl.BlockSpec((1, 1, br), lambda i: (out_map(i)[0], 0, 0)),
        pl.BlockSpec((1, d), lambda i: (0, 0)),
        pl.BlockSpec((1, d), lambda i: (0, 0)),
    ]
    args = [tok, pos_table, seg_table, sid3, gamma2d, beta2d]
    aliases = {}
    if prev is not None:
        in_specs.append(pl.BlockSpec(memory_space=pl.ANY))
        args.append(prev)
        aliases = {6: 0}

    return pl.pallas_call(
        body,
        grid=(nblk,),
        in_specs=in_specs,
        out_specs=pl.BlockSpec((br, d), out_map),
        out_shape=jax.ShapeDtypeStruct((total_n, d), jnp.float32),
        input_output_aliases=aliases,
    )(*args)


def kernel(token_ids, segment_ids, input_ids, token_table, segment_table,
           position_table, ln_gamma, ln_beta):
    b, s = input_ids.shape
    d = token_table.shape[1]
    n = b * s
    br = _TC_BLOCK_ROWS

    tids = token_ids.astype(jnp.int32)
    seg_pad = jnp.pad(segment_table, ((0, 8 - segment_table.shape[0]), (0, 0)))
    sid3 = segment_ids.astype(jnp.int32).reshape(n // br, 1, br)
    gamma2d = ln_gamma.reshape(1, d)
    beta2d = ln_beta.reshape(1, d)

    s_chunk = s // _NPARTS
    toks = [
        _sc_gather_part(
            token_table,
            lax.slice(tids, (0, h * s_chunk), (b, (h + 1) * s_chunk)))
        for h in range(_NPARTS)
    ]
    out = None
    for h in range(_NPARTS):
        out = _tc_add_layernorm_part(
            toks[h], position_table, seg_pad, sid3, gamma2d, beta2d,
            1e-5, out, h, s_chunk, n, b,
        )
    return out.reshape(b, s, d)


# speedup vs baseline: 1.1726x; 1.0687x over previous
import functools

import jax
import jax.numpy as jnp
from jax import lax
from jax.experimental import pallas as pl
from jax.experimental.pallas import tpu as pltpu
from jax.experimental.pallas import tpu_sc as plsc

_NC = 2
_NS = 16
_NW = _NC * _NS

_NPARTS = 4
_TC_BLOCK_ROWS = 256
_GATHER_CHUNK = 32


def _sc_gather_part(table, tids_part):
    b_sz, s_chunk = tids_part.shape
    d = table.shape[1]
    n_p = b_sz * s_chunk
    rpw = n_p // _NW
    wpb = _NW // b_sz
    ch = min(_GATHER_CHUNK, rpw)
    nch = rpw // ch
    assert rpw * _NW == n_p and ch * nch == rpw

    mesh = plsc.VectorSubcoreMesh(
        core_axis_name="c", subcore_axis_name="s",
        num_cores=_NC, num_subcores=_NS,
    )

    @functools.partial(
        pl.kernel,
        mesh=mesh,
        out_type=jax.ShapeDtypeStruct((n_p, d), jnp.float32),
        scratch_types=[
            pltpu.VMEM((rpw,), jnp.int32),
            pltpu.VMEM((ch, d), jnp.float32),
            pltpu.VMEM((ch, d), jnp.float32),
            pltpu.SemaphoreType.DMA,
            pltpu.SemaphoreType.DMA,
            pltpu.SemaphoreType.DMA,
            pltpu.SemaphoreType.DMA,
        ],
    )
    def k(table_hbm, tids_hbm, out_hbm, idx_v, rows0, rows1, g0, g1, s0, s1):
        wid = lax.axis_index("s") * _NC + lax.axis_index("c")
        bi = wid // wpb
        si = (wid % wpb) * rpw
        base = wid * rpw
        pltpu.sync_copy(tids_hbm.at[bi, pl.ds(si, rpw)], idx_v)
        rows = (rows0, rows1)
        gsem = (g0, g1)
        ssem = (s0, s1)

        def gather(ci):
            b = ci % 2
            return pltpu.async_copy(
                table_hbm.at[idx_v.at[pl.ds(ci * ch, ch)]], rows[b], gsem[b])

        def store(ci):
            b = ci % 2
            return pltpu.async_copy(
                rows[b], out_hbm.at[pl.ds(base + ci * ch, ch)], ssem[b])

        gcp = [None, None]
        scp = [None, None]
        gcp[0] = gather(0)
        for ci in range(nch):
            b = ci % 2
            nb = (ci + 1) % 2
            if ci + 1 < nch:
                if scp[nb] is not None:
                    scp[nb].wait()
                gcp[nb] = gather(ci + 1)
            gcp[b].wait()
            scp[b] = store(ci)
        for cp in scp:
            if cp is not None:
                cp.wait()

    return k(table, tids_part)


def _tc_add_layernorm_part(tok, pos_table, seg_table, sid3, gamma2d, beta2d,
                           eps, prev, h, s_chunk, total_n, b_sz):
    n_p, d = tok.shape
    s_len = pos_table.shape[0]
    br = s_chunk
    assert n_p % br == 0
    spb = s_len // br
    nblk = n_p // br

    def out_map(i):
        return (i * spb + h, 0)

    def body(tok_ref, pos_ref, seg_ref, sid_ref, g_ref, b_ref, *rest):
        o_ref = rest[-1]
        s0 = seg_ref[0, :]
        sd = seg_ref[1, :] - s0
        sid_col = sid_ref[0, 0, :].astype(jnp.float32).reshape(br, 1)
        x = tok_ref[...] + pos_ref[...] + s0[None, :] + sid_col * sd[None, :]
        mean = jnp.mean(x, axis=-1, keepdims=True)
        xc = x - mean
        var = jnp.mean(xc * xc, axis=-1, keepdims=True)
        inv = lax.rsqrt(var + eps)
        o_ref[...] = xc * inv * g_ref[...] + b_ref[...]

    in_specs = [
        pl.BlockSpec((br, d), lambda i: (i, 0)),
        pl.BlockSpec((br, d), lambda i: (h, 0)),
        pl.BlockSpec((8, d), lambda i: (0, 0)),
        pl.BlockSpec((1, 1, br), lambda i: (out_map(i)[0], 0, 0)),
        pl.BlockSpec((1, d), lambda i: (0, 0)),
        pl.BlockSpec((1, d), lambda i: (0, 0)),
    ]
    args = [tok, pos_table, seg_table, sid3, gamma2d, beta2d]
    aliases = {}
    if prev is not None:
        in_specs.append(pl.BlockSpec(memory_space=pl.ANY))
        args.append(prev)
        aliases = {6: 0}

    return pl.pallas_call(
        body,
        grid=(nblk,),
        in_specs=in_specs,
        out_specs=pl.BlockSpec((br, d), out_map),
        out_shape=jax.ShapeDtypeStruct((total_n, d), jnp.float32),
        input_output_aliases=aliases,
    )(*args)


def kernel(token_ids, segment_ids, input_ids, token_table, segment_table,
           position_table, ln_gamma, ln_beta):
    b, s = input_ids.shape
    d = token_table.shape[1]
    n = b * s
    s_chunk = s // _NPARTS

    tids = token_ids.astype(jnp.int32)
    seg_pad = jnp.pad(segment_table, ((0, 8 - segment_table.shape[0]), (0, 0)))
    sid3 = segment_ids.astype(jnp.int32).reshape(n // s_chunk, 1, s_chunk)
    gamma2d = ln_gamma.reshape(1, d)
    beta2d = ln_beta.reshape(1, d)

    toks = [
        _sc_gather_part(
            token_table,
            lax.slice(tids, (0, h * s_chunk), (b, (h + 1) * s_chunk)))
        for h in range(_NPARTS)
    ]
    out = None
    for h in range(_NPARTS):
        out = _tc_add_layernorm_part(
            toks[h], position_table, seg_pad, sid3, gamma2d, beta2d,
            1e-5, out, h, s_chunk, n, b,
        )
    return out.reshape(b, s, d)
